# Initial kernel scaffold; baseline (speedup 1.0000x reference)
import jax
import jax.numpy as jnp
from jax.experimental import pallas as pl


def kernel(coords, codebook, indices):
    raise NotImplementedError("write your pallas kernel here")


# speedup vs baseline: 34.0095x; 34.0095x over previous
import functools

import jax
import jax.numpy as jnp
from jax import lax
from jax.experimental import pallas as pl
from jax.experimental.pallas import tpu as pltpu
from jax.experimental.pallas import tpu_sc as plsc

R = 262144
NCODES = 64
CODE = 4
N = 2097152
QUARTER = R // 4

_TC_BLOCK = 2048


def _code_body(idx_ref, out_ref):
    x = idx_ref[...]
    m = jnp.max(x, axis=-1, keepdims=True)
    it = lax.broadcasted_iota(jnp.int32, x.shape, 2)
    a = jnp.min(jnp.where(x == m, it, NCODES), axis=-1)
    p = a[0] | (a[1] << 8) | (a[2] << 16) | (a[3] << 24)
    out_ref[...] = p


_codes_tc = pl.pallas_call(
    _code_body,
    grid=(QUARTER // _TC_BLOCK,),
    in_specs=[pl.BlockSpec((4, _TC_BLOCK, NCODES), lambda i: (0, i, 0))],
    out_specs=pl.BlockSpec((_TC_BLOCK,), lambda i: (i,)),
    out_shape=jax.ShapeDtypeStruct((QUARTER,), jnp.int32),
)

_NW = 32
_PER_W = N // _NW
_CHUNK = 2048
_SCALE = 0.5 * (R - 1)

_mesh = plsc.VectorSubcoreMesh(core_axis_name="c", subcore_axis_name="s")


@functools.partial(
    pl.kernel,
    mesh=_mesh,
    out_type=jax.ShapeDtypeStruct((N * CODE,), jnp.float32),
    compiler_params=pltpu.CompilerParams(needs_layout_passes=False),
    scratch_types=[
        pltpu.VMEM((QUARTER,), jnp.int32),
        pltpu.VMEM((NCODES * CODE,), jnp.float32),
        pltpu.VMEM((NCODES,), jnp.float32),
        pltpu.VMEM((NCODES,), jnp.float32),
        pltpu.VMEM((NCODES,), jnp.float32),
        pltpu.VMEM((NCODES,), jnp.float32),
        pltpu.VMEM((_CHUNK,), jnp.float32),
        pltpu.VMEM((_CHUNK * CODE,), jnp.float32),
    ],
)
def _interp_sc(coords_hbm, packed_hbm, cb_hbm, out_hbm,
               packed_v, cbflat_v, cb0, cb1, cb2, cb3, cbuf, obuf):
    wid = lax.axis_index("s") * 2 + lax.axis_index("c")
    base = wid * _PER_W

    pltpu.sync_copy(packed_hbm, packed_v)
    pltpu.sync_copy(cb_hbm, cbflat_v)

    it = lax.iota(jnp.int32, 16)
    cbcols = (cb0, cb1, cb2, cb3)
    for k in range(CODE):
        for s in range(NCODES // 16):
            rows = it + s * 16
            cbcols[k][pl.ds(s * 16, 16)] = plsc.load_gather(
                cbflat_v, [rows * CODE + k])

    it4 = it << 2

    def step(s, carry):
        j0 = s * 16
        c = (cbuf[pl.ds(j0, 16)] + 1.0) * _SCALE
        li = jnp.minimum(c.astype(jnp.int32), R - 2)
        w = c - li.astype(jnp.float32)
        ri = li + 1
        pw_l = plsc.load_gather(packed_v, [li & 0xFFFF])
        code_l = (pw_l >> ((li >> 16) << 3)) & 63
        pw_r = plsc.load_gather(packed_v, [ri & 0xFFFF])
        code_r = (pw_r >> ((ri >> 16) << 3)) & 63
        o_base = it4 + (j0 << 2)
        for k in range(CODE):
            lv = plsc.load_gather(cbcols[k], [code_l])
            rv = plsc.load_gather(cbcols[k], [code_r])
            plsc.store_scatter(obuf, [o_base + k], lv + w * (rv - lv))
        return carry

    def chunk(ci, carry):
        off = base + ci * _CHUNK
        pltpu.sync_copy(coords_hbm.at[pl.ds(off, _CHUNK)], cbuf)
        lax.fori_loop(0, _CHUNK // 16, step, 0)
        pltpu.sync_copy(obuf, out_hbm.at[pl.ds(off * CODE, _CHUNK * CODE)])
        return carry

    lax.fori_loop(0, _PER_W // _CHUNK, chunk, 0)


def kernel(coords, codebook, indices):
    packed = _codes_tc(indices.reshape(4, QUARTER, NCODES))
    out = _interp_sc(coords, packed, codebook.reshape(-1))
    return out.reshape(N, CODE)


# speedup vs baseline: 201.9756x; 5.9388x over previous
import functools

import jax
import jax.numpy as jnp
from jax import lax
from jax.experimental import pallas as pl
from jax.experimental.pallas import tpu as pltpu
from jax.experimental.pallas import tpu_sc as plsc

R = 262144
NCODES = 64
CODE = 4
N = 2097152
QUARTER = R // 4

_TC_BLOCK = 2048
_TC_GRID = QUARTER // _TC_BLOCK


def _code_body(x0, x1, x2, x3, out_ref):
    word = None
    for q, ref in enumerate((x0, x1, x2, x3)):
        x = ref[...]
        m = jnp.max(x, axis=0)
        it = lax.broadcasted_iota(jnp.int32, x.shape, 0)
        a = jnp.min(jnp.where(x == m[None, :], it, NCODES), axis=0)
        part = a << (8 * q)
        word = part if word is None else word | part
    out_ref[...] = word


_codes_tc = pl.pallas_call(
    _code_body,
    grid=(_TC_GRID,),
    in_specs=[
        pl.BlockSpec((NCODES, _TC_BLOCK), lambda i, q=q: (0, q * _TC_GRID + i))
        for q in range(4)
    ],
    out_specs=pl.BlockSpec((_TC_BLOCK,), lambda i: (i,)),
    out_shape=jax.ShapeDtypeStruct((QUARTER,), jnp.int32),
)

_NW = 32
_PER_W = N // _NW
_CHUNK = 2048
_SCALE = 0.5 * (R - 1)

_mesh = plsc.VectorSubcoreMesh(core_axis_name="c", subcore_axis_name="s")


@functools.partial(
    pl.kernel,
    mesh=_mesh,
    out_type=jax.ShapeDtypeStruct((N * CODE,), jnp.float32),
    compiler_params=pltpu.CompilerParams(needs_layout_passes=False),
    scratch_types=[
        pltpu.VMEM((QUARTER,), jnp.int32),
        pltpu.VMEM((NCODES * CODE,), jnp.float32),
        pltpu.VMEM((NCODES,), jnp.float32),
        pltpu.VMEM((NCODES,), jnp.float32),
        pltpu.VMEM((NCODES,), jnp.float32),
        pltpu.VMEM((NCODES,), jnp.float32),
        pltpu.VMEM((_CHUNK,), jnp.float32),
        pltpu.VMEM((_CHUNK * CODE,), jnp.float32),
    ],
)
def _interp_sc(coords_hbm, packed_hbm, cb_hbm, out_hbm,
               packed_v, cbflat_v, cb0, cb1, cb2, cb3, cbuf, obuf):
    wid = lax.axis_index("s") * 2 + lax.axis_index("c")
    base = wid * _PER_W

    pltpu.sync_copy(packed_hbm, packed_v)
    pltpu.sync_copy(cb_hbm, cbflat_v)

    cbcols = (cb0, cb1, cb2, cb3)
    for k in range(CODE):
        for s in range(NCODES // 16):
            cbcols[k][pl.ds(s * 16, 16)] = cbflat_v[pl.ds(k * NCODES + s * 16, 16)]

    def step(s, carry):
        j0 = s * 16
        c = (cbuf[pl.ds(j0, 16)] + 1.0) * _SCALE
        li = jnp.minimum(c.astype(jnp.int32), R - 2)
        w = c - li.astype(jnp.float32)
        ri = li + 1
        pw_l = plsc.load_gather(packed_v, [li & 0xFFFF])
        code_l = (pw_l >> ((li >> 16) << 3)) & 63
        pw_r = plsc.load_gather(packed_v, [ri & 0xFFFF])
        code_r = (pw_r >> ((ri >> 16) << 3)) & 63
        off0 = ((j0 >> 7) << 9) + (j0 & 127)
        for k in range(CODE):
            lv = plsc.load_gather(cbcols[k], [code_l])
            rv = plsc.load_gather(cbcols[k], [code_r])
            obuf[pl.ds(off0 + k * 128, 16)] = lv + w * (rv - lv)
        return carry

    def chunk(ci, carry):
        off = base + ci * _CHUNK
        pltpu.sync_copy(coords_hbm.at[pl.ds(off, _CHUNK)], cbuf)
        lax.fori_loop(0, _CHUNK // 16, step, 0)
        pltpu.sync_copy(obuf, out_hbm.at[pl.ds(off * CODE, _CHUNK * CODE)])
        return carry

    lax.fori_loop(0, _PER_W // _CHUNK, chunk, 0)


def kernel(coords, codebook, indices):
    packed = _codes_tc(*([indices.T] * 4))
    flat = _interp_sc(coords, packed, codebook.T.reshape(-1))
    return flat.reshape(N // 128, CODE, 128).transpose(0, 2, 1).reshape(N, CODE)


# speedup vs baseline: 525.7833x; 2.6032x over previous
import functools

import jax
import jax.numpy as jnp
from jax import lax
from jax.experimental import pallas as pl
from jax.experimental.pallas import tpu as pltpu
from jax.experimental.pallas import tpu_sc as plsc

R = 262144
NCODES = 64
CODE = 4
N = 2097152
QUARTER = R // 4

_TC_BLOCK = 2048
_TC_GRID = QUARTER // _TC_BLOCK


def _code_body(x0, x1, x2, x3, out_ref):
    word = None
    for q, ref in enumerate((x0, x1, x2, x3)):
        x = ref[...]
        m = jnp.max(x, axis=0)
        it = lax.broadcasted_iota(jnp.int32, x.shape, 0)
        a = jnp.min(jnp.where(x == m[None, :], it, NCODES), axis=0)
        part = a << (8 * q)
        word = part if word is None else word | part
    out_ref[...] = word


_codes_tc = pl.pallas_call(
    _code_body,
    grid=(_TC_GRID,),
    in_specs=[
        pl.BlockSpec((NCODES, _TC_BLOCK), lambda i, q=q: (0, q * _TC_GRID + i))
        for q in range(4)
    ],
    out_specs=pl.BlockSpec((_TC_BLOCK,), lambda i: (i,)),
    out_shape=jax.ShapeDtypeStruct((QUARTER,), jnp.int32),
)

_NW = 32
_PER_W = N // _NW
_CHUNK = 4096
_NCH = _PER_W // _CHUNK
_SCALE = 0.5 * (R - 1)

_mesh = plsc.VectorSubcoreMesh(core_axis_name="c", subcore_axis_name="s")


@functools.partial(
    pl.kernel,
    mesh=_mesh,
    out_type=jax.ShapeDtypeStruct((N * CODE,), jnp.float32),
    compiler_params=pltpu.CompilerParams(needs_layout_passes=False),
    scratch_types=[
        pltpu.VMEM((QUARTER,), jnp.int32),
        pltpu.VMEM((NCODES * CODE,), jnp.float32),
        pltpu.VMEM((NCODES,), jnp.float32),
        pltpu.VMEM((NCODES,), jnp.float32),
        pltpu.VMEM((NCODES,), jnp.float32),
        pltpu.VMEM((NCODES,), jnp.float32),
        pltpu.VMEM((_CHUNK,), jnp.float32),
        pltpu.VMEM((_CHUNK,), jnp.float32),
        pltpu.VMEM((_CHUNK * CODE,), jnp.float32),
        pltpu.VMEM((_CHUNK * CODE,), jnp.float32),
        pltpu.SemaphoreType.DMA,
        pltpu.SemaphoreType.DMA,
        pltpu.SemaphoreType.DMA,
        pltpu.SemaphoreType.DMA,
    ],
)
def _interp_sc(coords_hbm, packed_hbm, cb_hbm, out_hbm,
               packed_v, cbflat_v, cb0, cb1, cb2, cb3,
               cbuf0, cbuf1, obuf0, obuf1, sin0, sin1, sout0, sout1):
    wid = lax.axis_index("s") * 2 + lax.axis_index("c")
    base = wid * _PER_W

    pltpu.async_copy(coords_hbm.at[pl.ds(base, _CHUNK)], cbuf0, sin0)
    pltpu.async_copy(coords_hbm.at[pl.ds(base + _CHUNK, _CHUNK)], cbuf1, sin1)
    pltpu.sync_copy(packed_hbm, packed_v)
    pltpu.sync_copy(cb_hbm, cbflat_v)

    cbcols = (cb0, cb1, cb2, cb3)
    for k in range(CODE):
        for s in range(NCODES // 16):
            cbcols[k][pl.ds(s * 16, 16)] = cbflat_v[pl.ds(k * NCODES + s * 16, 16)]

    def compute(cbuf, obuf):
        @plsc.parallel_loop(0, _CHUNK // 16, unroll=4)
        def _(s):
            j0 = s * 16
            c = (cbuf[pl.ds(j0, 16)] + 1.0) * _SCALE
            li = jnp.minimum(c.astype(jnp.int32), R - 2)
            w = c - li.astype(jnp.float32)
            ri = li + 1
            pw_l = plsc.load_gather(packed_v, [li & 0xFFFF])
            code_l = (pw_l >> ((li >> 16) << 3)) & 63
            pw_r = plsc.load_gather(packed_v, [ri & 0xFFFF])
            code_r = (pw_r >> ((ri >> 16) << 3)) & 63
            off0 = ((j0 >> 7) << 9) + (j0 & 127)
            for k in range(CODE):
                lv = plsc.load_gather(cbcols[k], [code_l])
                rv = plsc.load_gather(cbcols[k], [code_r])
                obuf[pl.ds(off0 + k * 128, 16)] = lv + w * (rv - lv)

    bufs = ((cbuf0, obuf0, sin0, sout0), (cbuf1, obuf1, sin1, sout1))

    def pair(i2, carry):
        for b in range(2):
            cbuf, obuf, sin, sout = bufs[b]
            ci = i2 * 2 + b
            off = base + ci * _CHUNK
            pltpu.make_async_copy(
                coords_hbm.at[pl.ds(base, _CHUNK)], cbuf, sin).wait()

            @pl.when(ci >= 2)
            def _():
                pltpu.make_async_copy(
                    obuf, out_hbm.at[pl.ds(base * CODE, _CHUNK * CODE)], sout
                ).wait()

            compute(cbuf, obuf)
            pltpu.async_copy(
                obuf, out_hbm.at[pl.ds(off * CODE, _CHUNK * CODE)], sout)

            @pl.when(ci + 2 < _NCH)
            def _():
                pltpu.async_copy(
                    coords_hbm.at[pl.ds(off + 2 * _CHUNK, _CHUNK)], cbuf, sin)
        return carry

    lax.fori_loop(0, _NCH // 2, pair, 0)

    for b in range(2):
        cbuf, obuf, sin, sout = bufs[b]
        pltpu.make_async_copy(
            obuf, out_hbm.at[pl.ds(base * CODE, _CHUNK * CODE)], sout).wait()


def kernel(coords, codebook, indices):
    packed = _codes_tc(*([indices.T] * 4))
    flat = _interp_sc(coords, packed, codebook.T.reshape(-1))
    return flat.reshape(N // 128, CODE, 128).transpose(0, 2, 1).reshape(N, CODE)
